# baseline (speedup 1.0000x reference)
import functools
import jax
import jax.numpy as jnp
from jax import lax
from jax.experimental import pallas as pl
from jax.experimental.pallas import tpu as pltpu
from jax.experimental.pallas import tpu_sc as plsc

N = 10000
E = 160000
D = 128
TILE = 128
NTILES = E // TILE
NC = 2
NS = 16
NW = NC * NS
NJMAX = -(-NTILES // NW)
ZCH = 80
NCH = N // ZCH
F32 = jnp.float32

_mesh = plsc.VectorSubcoreMesh(core_axis_name="c", subcore_axis_name="s")


def _dot(a, b):
    return jnp.dot(a, b, preferred_element_type=F32)


def _zero_acc(z_v, acc, sid):
    @pl.loop(sid, NCH, step=NS)
    def _(k):
        pltpu.sync_copy(z_v.at[pl.ds(0, ZCH)], acc.at[pl.ds(k * ZCH, ZCH)])


def _dump_acc(acc, out_hbm, cid, sid):
    @pl.loop(sid, NCH, step=NS)
    def _(k):
        pltpu.sync_copy(acc.at[pl.ds(k * ZCH, ZCH)],
                        out_hbm.at[cid].at[pl.ds(k * ZCH, ZCH)])


def _ring(wid, nj, njmax, idx_load, idx_wait, data_start, data_wait, post,
          scat):
    idx_load(0, 0)

    @pl.loop(0, njmax, step=2)
    def _(jb):
        for b in (0, 1):
            j = jb + b

            @pl.when(j < nj)
            def _(j=j, b=b):
                idx_wait(b)
                data_start(j, b)

                @pl.when(j >= 1)
                def _():
                    scat(1 - b)

                @pl.when(j + 1 < nj)
                def _():
                    idx_load(j + 1, 1 - b)

                data_wait(b)
                post(j, b)

    @pl.when(nj % 2 == 1)
    def _():
        scat(0)

    @pl.when(nj % 2 == 0)
    def _():
        scat(1)


def _nop(j, b):
    pass


def _k2_call(dst2, src2, x, ef, zrows, ones128):
    outs = (jax.ShapeDtypeStruct((NC, N, D), F32),
            jax.ShapeDtypeStruct((NC, N, D), F32),
            jax.ShapeDtypeStruct((NC, N, D), F32))

    @functools.partial(
        pl.kernel, mesh=_mesh, out_type=outs,
        scratch_types=[
            pltpu.VMEM((TILE,), jnp.int32), pltpu.VMEM((TILE,), jnp.int32),
            pltpu.VMEM((TILE,), jnp.int32), pltpu.VMEM((TILE,), jnp.int32),
            pltpu.VMEM((TILE, D), F32), pltpu.VMEM((TILE, D), F32),
            pltpu.VMEM_SHARED((N, D), F32),
        ] + [pltpu.SemaphoreType.DMA] * 4,
    )
    def k2(dst_h, src_h, x_h, ef_h, zr_h, on_h,
           sx_o, sef_o, cnt_o,
           idxd0, idxd1, idxs0, idxs1, rows0, rows1, accD,
           si0, si1, sg0, sg1):
        cid = lax.axis_index("c")
        sid = lax.axis_index("s")
        wid = sid * NC + cid
        nj = jnp.where(wid < NTILES - NW * (NJMAX - 1), NJMAX, NJMAX - 1)
        idxd = (idxd0, idxd1)
        idxs = (idxs0, idxs1)
        rows = (rows0, rows1)
        sI = (si0, si1)
        sG = (sg0, sg1)

        def idxw(b):
            pltpu.make_async_copy(dst_h.at[0], idxd[b], sI[b]).wait()

        def scat(b):
            pltpu.sync_copy(rows[b], accD.at[idxd[b]], add=True)

        pltpu.sync_copy(zr_h, rows0)
        _zero_acc(rows0, accD, sid)
        plsc.subcore_barrier()

        def a_il(j, b):
            t = wid + j * NW
            pltpu.make_async_copy(dst_h.at[t], idxd[b], sI[b]).start()
            pltpu.make_async_copy(src_h.at[t], idxs[b], sI[b]).start()

        def a_iw(b):
            idxw(b)
            pltpu.make_async_copy(src_h.at[0], idxs[b], sI[b]).wait()

        def a_ds(j, b):
            pltpu.make_async_copy(x_h.at[idxs[b]], rows[b], sG[b]).start()

        def a_dw(b):
            pltpu.make_async_copy(x_h.at[idxs[b]], rows[b], sG[b]).wait()

        _ring(wid, nj, NJMAX, a_il, a_iw, a_ds, a_dw, _nop, scat)
        plsc.subcore_barrier()
        _dump_acc(accD, sx_o, cid, sid)
        pltpu.sync_copy(zr_h, rows0)
        _zero_acc(rows0, accD, sid)
        plsc.subcore_barrier()

        def b_il(j, b):
            pltpu.make_async_copy(dst_h.at[wid + j * NW], idxd[b], sI[b]).start()

        def b_ds(j, b):
            t = wid + j * NW
            pltpu.make_async_copy(ef_h.at[pl.ds(t * TILE, TILE)], rows[b],
                                  sG[b]).start()

        def b_dw(b):
            pltpu.make_async_copy(ef_h.at[pl.ds(0, TILE)], rows[b],
                                  sG[b]).wait()

        _ring(wid, nj, NJMAX, b_il, idxw, b_ds, b_dw, _nop, scat)
        plsc.subcore_barrier()
        _dump_acc(accD, sef_o, cid, sid)
        pltpu.sync_copy(zr_h, rows1)
        _zero_acc(rows1, accD, sid)
        pltpu.sync_copy(on_h, rows0)
        plsc.subcore_barrier()

        def c_scat(b):
            pltpu.sync_copy(rows0, accD.at[idxd[b]], add=True)

        _ring(wid, nj, NJMAX, b_il, idxw, _nop, lambda b: None, _nop, c_scat)
        plsc.subcore_barrier()
        _dump_acc(accD, cnt_o, cid, sid)

    return k2(dst2, src2, x, ef, zrows, ones128)


def _k3_call(dst2, data, zrows):
    @functools.partial(
        pl.kernel, mesh=_mesh,
        out_type=jax.ShapeDtypeStruct((NC, N, D), F32),
        scratch_types=[
            pltpu.VMEM((TILE,), jnp.int32), pltpu.VMEM((TILE,), jnp.int32),
            pltpu.VMEM((TILE, D), F32), pltpu.VMEM((TILE, D), F32),
            pltpu.VMEM_SHARED((N, D), F32),
        ] + [pltpu.SemaphoreType.DMA] * 4,
    )
    def k3(dst_h, data_h, zr_h, out_o,
           idxd0, idxd1, rows0, rows1, acc, si0, si1, sg0, sg1):
        cid = lax.axis_index("c")
        sid = lax.axis_index("s")
        wid = sid * NC + cid
        nj = jnp.where(wid < NTILES - NW * (NJMAX - 1), NJMAX, NJMAX - 1)
        idxd = (idxd0, idxd1)
        rows = (rows0, rows1)
        sI = (si0, si1)
        sG = (sg0, sg1)

        pltpu.sync_copy(zr_h, rows0)
        _zero_acc(rows0, acc, sid)
        plsc.subcore_barrier()

        def il(j, b):
            pltpu.make_async_copy(dst_h.at[wid + j * NW], idxd[b], sI[b]).start()

        def iw(b):
            pltpu.make_async_copy(dst_h.at[0], idxd[b], sI[b]).wait()

        def ds_(j, b):
            t = wid + j * NW
            pltpu.make_async_copy(data_h.at[pl.ds(t * TILE, TILE)], rows[b],
                                  sG[b]).start()

        def dw(b):
            pltpu.make_async_copy(data_h.at[pl.ds(0, TILE)], rows[b],
                                  sG[b]).wait()

        def scat(b):
            pltpu.sync_copy(rows[b], acc.at[idxd[b]], add=True)

        _ring(wid, nj, NJMAX, il, iw, ds_, dw, _nop, scat)
        plsc.subcore_barrier()
        _dump_acc(acc, out_o, cid, sid)

    return k3(dst2, data, zrows)


def _k7_call(dst2, src2, table, zrows):
    @functools.partial(
        pl.kernel, mesh=_mesh,
        out_type=jax.ShapeDtypeStruct((NC, N, D), F32),
        scratch_types=[
            pltpu.VMEM((TILE,), jnp.int32), pltpu.VMEM((TILE,), jnp.int32),
            pltpu.VMEM((TILE,), jnp.int32), pltpu.VMEM((TILE,), jnp.int32),
            pltpu.VMEM((TILE, D), F32), pltpu.VMEM((TILE, D), F32),
            pltpu.VMEM_SHARED((N, D), F32),
        ] + [pltpu.SemaphoreType.DMA] * 4,
    )
    def k7(dst_h, src_h, tab_h, zr_h, out_o,
           idxd0, idxd1, idxs0, idxs1, rows0, rows1, acc,
           si0, si1, sg0, sg1):
        cid = lax.axis_index("c")
        sid = lax.axis_index("s")
        wid = sid * NC + cid
        nj = jnp.where(wid < NTILES - NW * (NJMAX - 1), NJMAX, NJMAX - 1)
        idxd = (idxd0, idxd1)
        idxs = (idxs0, idxs1)
        rows = (rows0, rows1)
        sI = (si0, si1)
        sG = (sg0, sg1)

        pltpu.sync_copy(zr_h, rows0)
        _zero_acc(rows0, acc, sid)
        plsc.subcore_barrier()

        def il(j, b):
            t = wid + j * NW
            pltpu.make_async_copy(dst_h.at[t], idxd[b], sI[b]).start()
            pltpu.make_async_copy(src_h.at[t], idxs[b], sI[b]).start()

        def iw(b):
            pltpu.make_async_copy(dst_h.at[0], idxd[b], sI[b]).wait()
            pltpu.make_async_copy(src_h.at[0], idxs[b], sI[b]).wait()

        def ds_(j, b):
            pltpu.make_async_copy(tab_h.at[idxs[b]], rows[b], sG[b]).start()

        def dw(b):
            pltpu.make_async_copy(tab_h.at[idxs[b]], rows[b], sG[b]).wait()

        def scat(b):
            pltpu.sync_copy(rows[b], acc.at[idxd[b]], add=True)

        _ring(wid, nj, NJMAX, il, iw, ds_, dw, _nop, scat)
        plsc.subcore_barrier()
        _dump_acc(acc, out_o, cid, sid)

    return k7(dst2, src2, table, zrows)


T5 = 64
NT5 = E // T5
NJ5 = -(-NT5 // NW)


def _k5_call(dst4, src4, a1, b1, c1, zrows):
    outs = (jax.ShapeDtypeStruct((NC, N, D), F32),
            jax.ShapeDtypeStruct((TILE, D), F32))

    @functools.partial(
        pl.kernel, mesh=_mesh, out_type=outs,
        scratch_types=[
            pltpu.VMEM((T5,), jnp.int32), pltpu.VMEM((T5,), jnp.int32),
            pltpu.VMEM((T5,), jnp.int32), pltpu.VMEM((T5,), jnp.int32),
            pltpu.VMEM((T5, D), F32), pltpu.VMEM((T5, D), F32),
            pltpu.VMEM((T5, D), F32), pltpu.VMEM((T5, D), F32),
            pltpu.VMEM((T5, D), F32), pltpu.VMEM((T5, D), F32),
            pltpu.VMEM_SHARED((N, D), F32),
        ] + [pltpu.SemaphoreType.DMA] * 4,
    )
    def k5(dst_h, src_h, a_h, b_h, c_h, zr_h, seres_o, e128_o,
           idxd0, idxd1, idxs0, idxs1, ra0, ra1, rb0, rb1, rc0, rc1, acc,
           si0, si1, sg0, sg1):
        cid = lax.axis_index("c")
        sid = lax.axis_index("s")
        wid = sid * NC + cid
        nj = jnp.where(wid < NT5 - NW * (NJ5 - 1), NJ5, NJ5 - 1)
        idxd = (idxd0, idxd1)
        idxs = (idxs0, idxs1)
        ra = (ra0, ra1)
        rb = (rb0, rb1)
        rc = (rc0, rc1)
        sI = (si0, si1)
        sG = (sg0, sg1)

        _zero_acc(zr_h, acc, sid)
        plsc.subcore_barrier()

        def il(j, b):
            t = wid + j * NW
            pltpu.make_async_copy(dst_h.at[t], idxd[b], sI[b]).start()
            pltpu.make_async_copy(src_h.at[t], idxs[b], sI[b]).start()

        def iw(b):
            pltpu.make_async_copy(dst_h.at[0], idxd[b], sI[b]).wait()
            pltpu.make_async_copy(src_h.at[0], idxs[b], sI[b]).wait()

        def ds_(j, b):
            t = wid + j * NW
            pltpu.make_async_copy(a_h.at[idxs[b]], ra[b], sG[b]).start()
            pltpu.make_async_copy(b_h.at[idxd[b]], rb[b], sG[b]).start()
            pltpu.make_async_copy(c_h.at[pl.ds(t * T5, T5)], rc[b],
                                  sG[b]).start()

        def dw(b):
            pltpu.make_async_copy(a_h.at[idxs[b]], ra[b], sG[b]).wait()
            pltpu.make_async_copy(b_h.at[idxd[b]], rb[b], sG[b]).wait()
            pltpu.make_async_copy(c_h.at[pl.ds(0, T5)], rc[b], sG[b]).wait()

        def post(j, b):
            rav, rbv, rcv = ra[b], rb[b], rc[b]

            @plsc.parallel_loop(0, T5, unroll=4)
            def _(i):
                for jj in range(D // 16):
                    sl = pl.ds(jj * 16, 16)
                    v = rav[i, sl] + rbv[i, sl] + rcv[i, sl]
                    rcv[i, sl] = jnp.maximum(v, 0.0)

            t = wid + j * NW

            @pl.when(t < 2)
            def _():
                pltpu.sync_copy(rcv, e128_o.at[pl.ds(t * T5, T5)])

        def scat(b):
            pltpu.sync_copy(rc[b], acc.at[idxd[b]], add=True)

        _ring(wid, nj, NJ5 + 1, il, iw, ds_, dw, post, scat)
        plsc.subcore_barrier()
        _dump_acc(acc, seres_o, cid, sid)

    return k5(dst4, src4, a1, b1, c1, zrows)


def _k1_call(ts, ef, wrow, te_W1T, te_b1row, Web1T, M1T, c1row):
    BE = 1600
    grid = (E // BE,)

    def body(ts_r, ef_r, w_r, w1t_r, b1_r, webt_r, m1t_r, c1c_r, h_o, c1_o):
        h = jnp.sin(ts_r[...] * w_r[...])
        h = jnp.maximum(_dot(h, w1t_r[...]) + b1_r[...], 0.0)
        h_o[...] = h
        c1_o[...] = _dot(ef_r[...], webt_r[...]) + _dot(h, m1t_r[...]) + c1c_r[...]

    rep = pl.BlockSpec((128, 128), lambda i: (0, 0))
    rrow = pl.BlockSpec((1, 128), lambda i: (0, 0))
    return pl.pallas_call(
        body,
        grid=grid,
        in_specs=[pl.BlockSpec((BE, 1), lambda i: (i, 0)),
                  pl.BlockSpec((BE, D), lambda i: (i, 0)),
                  rrow, rep, rrow, rep, rep, rrow],
        out_specs=[pl.BlockSpec((BE, D), lambda i: (i, 0)),
                   pl.BlockSpec((BE, D), lambda i: (i, 0))],
        out_shape=[jax.ShapeDtypeStruct((E, D), F32),
                   jax.ShapeDtypeStruct((E, D), F32)],
    )(ts, ef, wrow, te_W1T, te_b1row, Web1T, M1T, c1row)


NB = 2000


def _k4_call(cnt2, sx2, sef2, sh2, x, WaT, WbT, McT, cconst, WdT, bnrow,
             WeaT, WedT):
    def body(cnt_r, sx_r, sef_r, sh_r, x_r, wa_r, wb_r, mc_r, cc_r, wd_r,
             bn_r, wea_r, wed_r, res_o, a_o, b_o):
        cnt = cnt_r[0, :, 0:1] + cnt_r[1, :, 0:1]
        invc = 1.0 / jnp.maximum(cnt, 1.0)
        ind = (cnt > 0.0).astype(F32)
        sx = (sx_r[0] + sx_r[1]) * invc
        sef = (sef_r[0] + sef_r[1]) * invc
        sh = (sh_r[0] + sh_r[1]) * invc
        nh = (_dot(sx, wa_r[...]) + _dot(sef, wb_r[...]) + _dot(sh, mc_r[...])
              + ind * cc_r[...] + _dot(x_r[...], wd_r[...]) + bn_r[...])
        res_o[...] = jnp.maximum(nh, 0.0)
        a_o[...] = _dot(nh, wea_r[...])
        b_o[...] = _dot(nh, wed_r[...])

    rep = pl.BlockSpec((D, D), lambda i: (0, 0))
    rrow = pl.BlockSpec((1, D), lambda i: (0, 0))
    half = pl.BlockSpec((2, NB, D), lambda i: (0, i, 0))
    nblk = pl.BlockSpec((NB, D), lambda i: (i, 0))
    outs = [jax.ShapeDtypeStruct((N, D), F32)] * 3
    return pl.pallas_call(
        body, grid=(N // NB,),
        in_specs=[half, half, half, half, nblk, rep, rep, rep, rrow, rep,
                  rrow, rep, rep],
        out_specs=[nblk, nblk, nblk],
        out_shape=outs)(
        cnt2, sx2, sef2, sh2, x, WaT, WbT, McT, cconst, WdT, bnrow, WeaT, WedT)


def _k8_call(cnt2, sres2, seres2, sh2, res1, WaT, WbT, McT, cconst, WdT,
             bnrow, WeaT, WedT, e128, Hh, Web2T, M2T, c2row, src_col, dst_col):
    def body(cnt_r, sr_r, se_r, sh_r, res_r, wa_r, wb_r, mc_r, cc_r, wd_r,
             bn_r, wea_r, wed_r, e128_r, hh_r, web_r, m2_r, c2c_r,
             srcc_r, dstc_r, out_o):
        i = pl.program_id(0)
        cnt = cnt_r[0, :, 0:1] + cnt_r[1, :, 0:1]
        invc = 1.0 / jnp.maximum(cnt, 1.0)
        ind = (cnt > 0.0).astype(F32)
        sr = (sr_r[0] + sr_r[1]) * invc
        se = (se_r[0] + se_r[1]) * invc
        sh = (sh_r[0] + sh_r[1]) * invc
        nh = (_dot(sr, wa_r[...]) + _dot(se, wb_r[...]) + _dot(sh, mc_r[...])
              + ind * cc_r[...] + _dot(res_r[...], wd_r[...]) + bn_r[...])
        a2 = _dot(nh, wea_r[...])
        b2 = _dot(nh, wed_r[...])
        io = lax.broadcasted_iota(jnp.int32, (TILE, NB), 1) + i * NB
        oh_s = (io == srcc_r[...]).astype(F32)
        oh_d = (io == dstc_r[...]).astype(F32)

        @pl.when(i == 0)
        def _():
            out_o[...] = (_dot(e128_r[...], web_r[...])
                          + _dot(hh_r[...], m2_r[...]) + c2c_r[...])

        out_o[...] += _dot(oh_s, a2) + _dot(oh_d, b2)

    rep = pl.BlockSpec((D, D), lambda i: (0, 0))
    rrow = pl.BlockSpec((1, D), lambda i: (0, 0))
    half = pl.BlockSpec((2, NB, D), lambda i: (0, i, 0))
    nblk = pl.BlockSpec((NB, D), lambda i: (i, 0))
    col = pl.BlockSpec((TILE, 1), lambda i: (0, 0))
    return pl.pallas_call(
        body, grid=(N // NB,),
        in_specs=[half, half, half, half, nblk, rep, rep, rep, rrow, rep,
                  rrow, rep, rep, rep, rep, rep, rep, rrow, col, col],
        out_specs=pl.BlockSpec((TILE, D), lambda i: (0, 0)),
        out_shape=jax.ShapeDtypeStruct((TILE, D), F32))(
        cnt2, sres2, seres2, sh2, res1, WaT, WbT, McT, cconst, WdT, bnrow,
        WeaT, WedT, e128, Hh, Web2T, M2T, c2row, src_col, dst_col)


def kernel(x, edge_index, edge_feat, ts, w, te_W1, te_b1, te_W2, te_b2,
           g1_Wn, g1_bn, g1_We, g1_be, g2_Wn, g2_bn, g2_We, g2_be):
    src = edge_index[0].astype(jnp.int32)
    dst = edge_index[1].astype(jnp.int32)
    src2 = src.reshape(NTILES, TILE)
    dst2 = dst.reshape(NTILES, TILE)

    def split(W):
        return W[:, :D], W[:, D:2 * D], W[:, 2 * D:3 * D], W[:, 3 * D:]

    Wa1, Wb1, Wc1, Wd1 = split(g1_Wn)
    Wea1, Web1, Wec1, Wed1 = split(g1_We)
    Wa2, Wb2, Wc2, Wd2 = split(g2_Wn)
    Wea2, Web2, Wec2, Wed2 = split(g2_We)
    row = lambda v: v.reshape(1, D)
    zrows = jnp.zeros((TILE, D), F32)
    ones128 = jnp.ones((TILE, D), F32)

    sx2, sef2, cnt2 = _k2_call(dst2, src2, x, edge_feat, zrows, ones128)
    H, c1 = _k1_call(ts, edge_feat, row(w), te_W1.T, row(te_b1), Web1.T,
                     (Wec1 @ te_W2).T, row(te_b2 @ Wec1.T + g1_be))
    sh2 = _k3_call(dst2, H, zrows)

    res1, a1, b1 = _k4_call(
        cnt2, sx2, sef2, sh2, x, Wa1.T, Wb1.T, (Wc1 @ te_W2).T,
        row(te_b2 @ Wc1.T), Wd1.T, row(g1_bn), Wea1.T, Wed1.T)

    seres2, e128 = _k5_call(dst.reshape(NT5, T5), src.reshape(NT5, T5),
                            a1, b1, c1, zrows)
    sres2 = _k7_call(dst2, src2, res1, zrows)

    out = _k8_call(
        cnt2, sres2, seres2, sh2, res1, Wa2.T, Wb2.T, (Wc2 @ te_W2).T,
        row(te_b2 @ Wc2.T), Wd2.T, row(g2_bn), Wea2.T, Wed2.T,
        e128, H[:TILE], Web2.T, (Wec2 @ te_W2).T,
        row(te_b2 @ Wec2.T + g2_be),
        src[:TILE].reshape(TILE, 1), dst[:TILE].reshape(TILE, 1))
    return out


# speedup vs baseline: 1.0297x; 1.0297x over previous
import functools
import jax
import jax.numpy as jnp
from jax import lax
from jax.experimental import pallas as pl
from jax.experimental.pallas import tpu as pltpu
from jax.experimental.pallas import tpu_sc as plsc

N = 10000
E = 160000
D = 128
TILE = 128
NTILES = E // TILE
NC = 2
NS = 16
NW = NC * NS
NJMAX = -(-NTILES // NW)
ZCH = 80
NCH = N // ZCH
F32 = jnp.float32

_mesh = plsc.VectorSubcoreMesh(core_axis_name="c", subcore_axis_name="s")


def _dot(a, b):
    return jnp.dot(a, b, preferred_element_type=F32)


def _zero_acc(z_v, acc, sid):
    @pl.loop(sid, NCH, step=NS)
    def _(k):
        pltpu.sync_copy(z_v.at[pl.ds(0, ZCH)], acc.at[pl.ds(k * ZCH, ZCH)])


def _dump_acc(acc, out_hbm, cid, sid):
    @pl.loop(sid, NCH, step=NS)
    def _(k):
        pltpu.sync_copy(acc.at[pl.ds(k * ZCH, ZCH)],
                        out_hbm.at[cid].at[pl.ds(k * ZCH, ZCH)])


def _ring(wid, nj, njmax, idx_load, idx_wait, data_start, data_wait, post,
          scat):
    idx_load(0, 0)

    @pl.loop(0, njmax, step=2)
    def _(jb):
        for b in (0, 1):
            j = jb + b

            @pl.when(j < nj)
            def _(j=j, b=b):
                idx_wait(b)
                data_start(j, b)

                @pl.when(j >= 1)
                def _():
                    scat(1 - b)

                @pl.when(j + 1 < nj)
                def _():
                    idx_load(j + 1, 1 - b)

                data_wait(b)
                post(j, b)

    @pl.when(nj % 2 == 1)
    def _():
        scat(0)

    @pl.when(nj % 2 == 0)
    def _():
        scat(1)


def _nop(j, b):
    pass


def _k2_call(dst2, src2, x, ef, zrows, ones128):
    outs = (jax.ShapeDtypeStruct((NC, N, D), F32),
            jax.ShapeDtypeStruct((NC, N, D), F32),
            jax.ShapeDtypeStruct((NC, N, D), F32))

    @functools.partial(
        pl.kernel, mesh=_mesh, out_type=outs,
        scratch_types=[
            pltpu.VMEM((TILE,), jnp.int32), pltpu.VMEM((TILE,), jnp.int32),
            pltpu.VMEM((TILE,), jnp.int32), pltpu.VMEM((TILE,), jnp.int32),
            pltpu.VMEM((TILE, D), F32), pltpu.VMEM((TILE, D), F32),
            pltpu.VMEM_SHARED((N, D), F32),
        ] + [pltpu.SemaphoreType.DMA] * 4,
    )
    def k2(dst_h, src_h, x_h, ef_h, zr_h, on_h,
           sx_o, sef_o, cnt_o,
           idxd0, idxd1, idxs0, idxs1, rows0, rows1, accD,
           si0, si1, sg0, sg1):
        cid = lax.axis_index("c")
        sid = lax.axis_index("s")
        wid = sid * NC + cid
        nj = jnp.where(wid < NTILES - NW * (NJMAX - 1), NJMAX, NJMAX - 1)
        idxd = (idxd0, idxd1)
        idxs = (idxs0, idxs1)
        rows = (rows0, rows1)
        sI = (si0, si1)
        sG = (sg0, sg1)

        def idxw(b):
            pltpu.make_async_copy(dst_h.at[0], idxd[b], sI[b]).wait()

        def scat(b):
            pltpu.sync_copy(rows[b], accD.at[idxd[b]], add=True)

        pltpu.sync_copy(zr_h, rows0)
        _zero_acc(rows0, accD, sid)
        plsc.subcore_barrier()

        def a_il(j, b):
            t = wid + j * NW
            pltpu.make_async_copy(dst_h.at[t], idxd[b], sI[b]).start()
            pltpu.make_async_copy(src_h.at[t], idxs[b], sI[b]).start()

        def a_iw(b):
            idxw(b)
            pltpu.make_async_copy(src_h.at[0], idxs[b], sI[b]).wait()

        def a_ds(j, b):
            pltpu.make_async_copy(x_h.at[idxs[b]], rows[b], sG[b]).start()

        def a_dw(b):
            pltpu.make_async_copy(x_h.at[idxs[b]], rows[b], sG[b]).wait()

        _ring(wid, nj, NJMAX, a_il, a_iw, a_ds, a_dw, _nop, scat)
        plsc.subcore_barrier()
        _dump_acc(accD, sx_o, cid, sid)
        pltpu.sync_copy(zr_h, rows0)
        _zero_acc(rows0, accD, sid)
        plsc.subcore_barrier()

        def b_il(j, b):
            pltpu.make_async_copy(dst_h.at[wid + j * NW], idxd[b], sI[b]).start()

        def b_ds(j, b):
            t = wid + j * NW
            pltpu.make_async_copy(ef_h.at[pl.ds(t * TILE, TILE)], rows[b],
                                  sG[b]).start()

        def b_dw(b):
            pltpu.make_async_copy(ef_h.at[pl.ds(0, TILE)], rows[b],
                                  sG[b]).wait()

        _ring(wid, nj, NJMAX, b_il, idxw, b_ds, b_dw, _nop, scat)
        plsc.subcore_barrier()
        _dump_acc(accD, sef_o, cid, sid)
        pltpu.sync_copy(zr_h, rows1)
        _zero_acc(rows1, accD, sid)
        pltpu.sync_copy(on_h, rows0)
        plsc.subcore_barrier()

        def c_scat(b):
            pltpu.sync_copy(rows0, accD.at[idxd[b]], add=True)

        _ring(wid, nj, NJMAX, b_il, idxw, _nop, lambda b: None, _nop, c_scat)
        plsc.subcore_barrier()
        _dump_acc(accD, cnt_o, cid, sid)

    return k2(dst2, src2, x, ef, zrows, ones128)


def _k3_call(dst2, data, zrows):
    @functools.partial(
        pl.kernel, mesh=_mesh,
        out_type=jax.ShapeDtypeStruct((NC, N, D), F32),
        scratch_types=[
            pltpu.VMEM((TILE,), jnp.int32), pltpu.VMEM((TILE,), jnp.int32),
            pltpu.VMEM((TILE, D), F32), pltpu.VMEM((TILE, D), F32),
            pltpu.VMEM_SHARED((N, D), F32),
        ] + [pltpu.SemaphoreType.DMA] * 4,
    )
    def k3(dst_h, data_h, zr_h, out_o,
           idxd0, idxd1, rows0, rows1, acc, si0, si1, sg0, sg1):
        cid = lax.axis_index("c")
        sid = lax.axis_index("s")
        wid = sid * NC + cid
        nj = jnp.where(wid < NTILES - NW * (NJMAX - 1), NJMAX, NJMAX - 1)
        idxd = (idxd0, idxd1)
        rows = (rows0, rows1)
        sI = (si0, si1)
        sG = (sg0, sg1)

        pltpu.sync_copy(zr_h, rows0)
        _zero_acc(rows0, acc, sid)
        plsc.subcore_barrier()

        def il(j, b):
            pltpu.make_async_copy(dst_h.at[wid + j * NW], idxd[b], sI[b]).start()

        def iw(b):
            pltpu.make_async_copy(dst_h.at[0], idxd[b], sI[b]).wait()

        def ds_(j, b):
            t = wid + j * NW
            pltpu.make_async_copy(data_h.at[pl.ds(t * TILE, TILE)], rows[b],
                                  sG[b]).start()

        def dw(b):
            pltpu.make_async_copy(data_h.at[pl.ds(0, TILE)], rows[b],
                                  sG[b]).wait()

        def scat(b):
            pltpu.sync_copy(rows[b], acc.at[idxd[b]], add=True)

        _ring(wid, nj, NJMAX, il, iw, ds_, dw, _nop, scat)
        plsc.subcore_barrier()
        _dump_acc(acc, out_o, cid, sid)

    return k3(dst2, data, zrows)


def _k7_call(dst2, src2, table, zrows):
    @functools.partial(
        pl.kernel, mesh=_mesh,
        out_type=jax.ShapeDtypeStruct((NC, N, D), F32),
        scratch_types=[
            pltpu.VMEM((TILE,), jnp.int32), pltpu.VMEM((TILE,), jnp.int32),
            pltpu.VMEM((TILE,), jnp.int32), pltpu.VMEM((TILE,), jnp.int32),
            pltpu.VMEM((TILE, D), F32), pltpu.VMEM((TILE, D), F32),
            pltpu.VMEM_SHARED((N, D), F32),
        ] + [pltpu.SemaphoreType.DMA] * 4,
    )
    def k7(dst_h, src_h, tab_h, zr_h, out_o,
           idxd0, idxd1, idxs0, idxs1, rows0, rows1, acc,
           si0, si1, sg0, sg1):
        cid = lax.axis_index("c")
        sid = lax.axis_index("s")
        wid = sid * NC + cid
        nj = jnp.where(wid < NTILES - NW * (NJMAX - 1), NJMAX, NJMAX - 1)
        idxd = (idxd0, idxd1)
        idxs = (idxs0, idxs1)
        rows = (rows0, rows1)
        sI = (si0, si1)
        sG = (sg0, sg1)

        pltpu.sync_copy(zr_h, rows0)
        _zero_acc(rows0, acc, sid)
        plsc.subcore_barrier()

        def il(j, b):
            t = wid + j * NW
            pltpu.make_async_copy(dst_h.at[t], idxd[b], sI[b]).start()
            pltpu.make_async_copy(src_h.at[t], idxs[b], sI[b]).start()

        def iw(b):
            pltpu.make_async_copy(dst_h.at[0], idxd[b], sI[b]).wait()
            pltpu.make_async_copy(src_h.at[0], idxs[b], sI[b]).wait()

        def ds_(j, b):
            pltpu.make_async_copy(tab_h.at[idxs[b]], rows[b], sG[b]).start()

        def dw(b):
            pltpu.make_async_copy(tab_h.at[idxs[b]], rows[b], sG[b]).wait()

        def scat(b):
            pltpu.sync_copy(rows[b], acc.at[idxd[b]], add=True)

        _ring(wid, nj, NJMAX, il, iw, ds_, dw, _nop, scat)
        plsc.subcore_barrier()
        _dump_acc(acc, out_o, cid, sid)

    return k7(dst2, src2, table, zrows)


def _k5_call(dst2, src2, a1, b1, c1, zrows):
    outs = (jax.ShapeDtypeStruct((NC, N, D), F32),
            jax.ShapeDtypeStruct((TILE, D), F32))

    @functools.partial(
        pl.kernel, mesh=_mesh, out_type=outs,
        scratch_types=[
            pltpu.VMEM((TILE,), jnp.int32), pltpu.VMEM((TILE,), jnp.int32),
            pltpu.VMEM((TILE,), jnp.int32), pltpu.VMEM((TILE,), jnp.int32),
            pltpu.VMEM((TILE, D), F32), pltpu.VMEM((TILE, D), F32),
            pltpu.VMEM((TILE, D), F32),
            pltpu.VMEM_SHARED((N, D), F32),
        ] + [pltpu.SemaphoreType.DMA] * 4,
    )
    def k5(dst_h, src_h, a_h, b_h, c_h, zr_h, seres_o, e128_o,
           idxd0, idxd1, idxs0, idxs1, ra, rb, rc, acc,
           si0, si1, sg0, sg1):
        cid = lax.axis_index("c")
        sid = lax.axis_index("s")
        wid = sid * NC + cid
        nj = jnp.where(wid < NTILES - NW * (NJMAX - 1), NJMAX, NJMAX - 1)
        idxd = (idxd0, idxd1)
        idxs = (idxs0, idxs1)
        sI = (si0, si1)
        sG = (sg0, sg1)

        _zero_acc(zr_h, acc, sid)
        plsc.subcore_barrier()

        def il(j, b):
            t = wid + j * NW
            pltpu.make_async_copy(dst_h.at[t], idxd[b], sI[b]).start()
            pltpu.make_async_copy(src_h.at[t], idxs[b], sI[b]).start()

        def iw(b):
            pltpu.make_async_copy(dst_h.at[0], idxd[b], sI[b]).wait()
            pltpu.make_async_copy(src_h.at[0], idxs[b], sI[b]).wait()

        def ds_(j, b):
            pltpu.make_async_copy(a_h.at[idxs[b]], ra, sG[b]).start()
            pltpu.make_async_copy(b_h.at[idxd[b]], rb, sG[b]).start()

        def dw(b):
            pass

        def post(j, b):
            t = wid + j * NW
            pltpu.sync_copy(c_h.at[pl.ds(t * TILE, TILE)], rc)
            pltpu.make_async_copy(a_h.at[idxs[b]], ra, sG[b]).wait()
            pltpu.make_async_copy(b_h.at[idxd[b]], rb, sG[b]).wait()

            @plsc.parallel_loop(0, TILE, unroll=4)
            def _(i):
                for jj in range(D // 16):
                    sl = pl.ds(jj * 16, 16)
                    v = ra[i, sl] + rb[i, sl] + rc[i, sl]
                    rc[i, sl] = jnp.maximum(v, 0.0)

            @pl.when(t == 0)
            def _():
                pltpu.sync_copy(rc, e128_o)

        def scat(b):
            pltpu.sync_copy(rc, acc.at[idxd[b]], add=True)

        _ring(wid, nj, NJMAX, il, iw, ds_, dw, post, scat)
        plsc.subcore_barrier()
        _dump_acc(acc, seres_o, cid, sid)

    return k5(dst2, src2, a1, b1, c1, zrows)


def _k1_call(ts, ef, wrow, te_W1T, te_b1row, Web1T, M1T, c1row):
    BE = 1600
    grid = (E // BE,)

    def body(ts_r, ef_r, w_r, w1t_r, b1_r, webt_r, m1t_r, c1c_r, h_o, c1_o):
        h = jnp.sin(ts_r[...] * w_r[...])
        h = jnp.maximum(_dot(h, w1t_r[...]) + b1_r[...], 0.0)
        h_o[...] = h
        c1_o[...] = _dot(ef_r[...], webt_r[...]) + _dot(h, m1t_r[...]) + c1c_r[...]

    rep = pl.BlockSpec((128, 128), lambda i: (0, 0))
    rrow = pl.BlockSpec((1, 128), lambda i: (0, 0))
    return pl.pallas_call(
        body,
        grid=grid,
        in_specs=[pl.BlockSpec((BE, 1), lambda i: (i, 0)),
                  pl.BlockSpec((BE, D), lambda i: (i, 0)),
                  rrow, rep, rrow, rep, rep, rrow],
        out_specs=[pl.BlockSpec((BE, D), lambda i: (i, 0)),
                   pl.BlockSpec((BE, D), lambda i: (i, 0))],
        out_shape=[jax.ShapeDtypeStruct((E, D), F32),
                   jax.ShapeDtypeStruct((E, D), F32)],
    )(ts, ef, wrow, te_W1T, te_b1row, Web1T, M1T, c1row)


NB = 2000


def _k4_call(cnt2, sx2, sef2, sh2, x, WaT, WbT, McT, cconst, WdT, bnrow,
             WeaT, WedT):
    def body(cnt_r, sx_r, sef_r, sh_r, x_r, wa_r, wb_r, mc_r, cc_r, wd_r,
             bn_r, wea_r, wed_r, res_o, a_o, b_o):
        cnt = cnt_r[0, :, 0:1] + cnt_r[1, :, 0:1]
        invc = 1.0 / jnp.maximum(cnt, 1.0)
        ind = (cnt > 0.0).astype(F32)
        sx = (sx_r[0] + sx_r[1]) * invc
        sef = (sef_r[0] + sef_r[1]) * invc
        sh = (sh_r[0] + sh_r[1]) * invc
        nh = (_dot(sx, wa_r[...]) + _dot(sef, wb_r[...]) + _dot(sh, mc_r[...])
              + ind * cc_r[...] + _dot(x_r[...], wd_r[...]) + bn_r[...])
        res_o[...] = jnp.maximum(nh, 0.0)
        a_o[...] = _dot(nh, wea_r[...])
        b_o[...] = _dot(nh, wed_r[...])

    rep = pl.BlockSpec((D, D), lambda i: (0, 0))
    rrow = pl.BlockSpec((1, D), lambda i: (0, 0))
    half = pl.BlockSpec((2, NB, D), lambda i: (0, i, 0))
    nblk = pl.BlockSpec((NB, D), lambda i: (i, 0))
    outs = [jax.ShapeDtypeStruct((N, D), F32)] * 3
    return pl.pallas_call(
        body, grid=(N // NB,),
        in_specs=[half, half, half, half, nblk, rep, rep, rep, rrow, rep,
                  rrow, rep, rep],
        out_specs=[nblk, nblk, nblk],
        out_shape=outs)(
        cnt2, sx2, sef2, sh2, x, WaT, WbT, McT, cconst, WdT, bnrow, WeaT, WedT)


def _k8_call(cnt2, sres2, seres2, sh2, res1, WaT, WbT, McT, cconst, WdT,
             bnrow, WeaT, WedT, e128, Hh, Web2T, M2T, c2row, src_col, dst_col):
    def body(cnt_r, sr_r, se_r, sh_r, res_r, wa_r, wb_r, mc_r, cc_r, wd_r,
             bn_r, wea_r, wed_r, e128_r, hh_r, web_r, m2_r, c2c_r,
             srcc_r, dstc_r, out_o):
        i = pl.program_id(0)
        cnt = cnt_r[0, :, 0:1] + cnt_r[1, :, 0:1]
        invc = 1.0 / jnp.maximum(cnt, 1.0)
        ind = (cnt > 0.0).astype(F32)
        sr = (sr_r[0] + sr_r[1]) * invc
        se = (se_r[0] + se_r[1]) * invc
        sh = (sh_r[0] + sh_r[1]) * invc
        nh = (_dot(sr, wa_r[...]) + _dot(se, wb_r[...]) + _dot(sh, mc_r[...])
              + ind * cc_r[...] + _dot(res_r[...], wd_r[...]) + bn_r[...])
        a2 = _dot(nh, wea_r[...])
        b2 = _dot(nh, wed_r[...])
        io = lax.broadcasted_iota(jnp.int32, (TILE, NB), 1) + i * NB
        oh_s = (io == srcc_r[...]).astype(F32)
        oh_d = (io == dstc_r[...]).astype(F32)

        @pl.when(i == 0)
        def _():
            out_o[...] = (_dot(e128_r[...], web_r[...])
                          + _dot(hh_r[...], m2_r[...]) + c2c_r[...])

        out_o[...] += _dot(oh_s, a2) + _dot(oh_d, b2)

    rep = pl.BlockSpec((D, D), lambda i: (0, 0))
    rrow = pl.BlockSpec((1, D), lambda i: (0, 0))
    half = pl.BlockSpec((2, NB, D), lambda i: (0, i, 0))
    nblk = pl.BlockSpec((NB, D), lambda i: (i, 0))
    col = pl.BlockSpec((TILE, 1), lambda i: (0, 0))
    return pl.pallas_call(
        body, grid=(N // NB,),
        in_specs=[half, half, half, half, nblk, rep, rep, rep, rrow, rep,
                  rrow, rep, rep, rep, rep, rep, rep, rrow, col, col],
        out_specs=pl.BlockSpec((TILE, D), lambda i: (0, 0)),
        out_shape=jax.ShapeDtypeStruct((TILE, D), F32))(
        cnt2, sres2, seres2, sh2, res1, WaT, WbT, McT, cconst, WdT, bnrow,
        WeaT, WedT, e128, Hh, Web2T, M2T, c2row, src_col, dst_col)


def kernel(x, edge_index, edge_feat, ts, w, te_W1, te_b1, te_W2, te_b2,
           g1_Wn, g1_bn, g1_We, g1_be, g2_Wn, g2_bn, g2_We, g2_be):
    src = edge_index[0].astype(jnp.int32)
    dst = edge_index[1].astype(jnp.int32)
    src2 = src.reshape(NTILES, TILE)
    dst2 = dst.reshape(NTILES, TILE)

    def split(W):
        return W[:, :D], W[:, D:2 * D], W[:, 2 * D:3 * D], W[:, 3 * D:]

    Wa1, Wb1, Wc1, Wd1 = split(g1_Wn)
    Wea1, Web1, Wec1, Wed1 = split(g1_We)
    Wa2, Wb2, Wc2, Wd2 = split(g2_Wn)
    Wea2, Web2, Wec2, Wed2 = split(g2_We)
    row = lambda v: v.reshape(1, D)
    zrows = jnp.zeros((TILE, D), F32)
    ones128 = jnp.ones((TILE, D), F32)

    sx2, sef2, cnt2 = _k2_call(dst2, src2, x, edge_feat, zrows, ones128)
    H, c1 = _k1_call(ts, edge_feat, row(w), te_W1.T, row(te_b1), Web1.T,
                     (Wec1 @ te_W2).T, row(te_b2 @ Wec1.T + g1_be))
    sh2 = _k3_call(dst2, H, zrows)

    res1, a1, b1 = _k4_call(
        cnt2, sx2, sef2, sh2, x, Wa1.T, Wb1.T, (Wc1 @ te_W2).T,
        row(te_b2 @ Wc1.T), Wd1.T, row(g1_bn), Wea1.T, Wed1.T)

    seres2, e128 = _k5_call(dst2, src2, a1, b1, c1, zrows)
    sres2 = _k7_call(dst2, src2, res1, zrows)

    out = _k8_call(
        cnt2, sres2, seres2, sh2, res1, Wa2.T, Wb2.T, (Wc2 @ te_W2).T,
        row(te_b2 @ Wc2.T), Wd2.T, row(g2_bn), Wea2.T, Wed2.T,
        e128, H[:TILE], Web2.T, (Wec2 @ te_W2).T,
        row(te_b2 @ Wec2.T + g2_be),
        src[:TILE].reshape(TILE, 1), dst[:TILE].reshape(TILE, 1))
    return out
